# baseline (speedup 1.0000x reference)
import functools

import jax
import jax.numpy as jnp
from jax import lax
from jax.experimental import pallas as pl
from jax.experimental.pallas import tpu as pltpu
from jax.experimental.pallas import tpu_sc as plsc

N = 10000
E = 320000
D_IN = 128
D_H = 64
D_OUT = 32

NC = 2
NS = 16
NW = NC * NS

C = 125
CH = 80
ROWS = E // C
ACC_N = 10112
RPT = ACC_N // NS
DEG_W = 16

_SC_PARAMS = dict(
    mesh=plsc.VectorSubcoreMesh(core_axis_name="c", subcore_axis_name="s"),
    compiler_params=pltpu.CompilerParams(use_tc_tiling_on_sc=False),
)


def _make_deg_kernel():
    @functools.partial(
        pl.kernel,
        out_type=jax.ShapeDtypeStruct((ACC_N, 128), jnp.float32),
        scratch_types=[
            pltpu.VMEM((CH, C), jnp.int32),
            pltpu.VMEM((C, DEG_W), jnp.float32),
            pltpu.VMEM_SHARED((ACC_N, DEG_W), jnp.float32),
            pltpu.SemaphoreType.DMA,
            pltpu.SemaphoreType.DMA,
        ],
        **_SC_PARAMS,
    )
    def deg_kernel(edge3d, ones_hbm, zeros_hbm, out_hbm, dst_v, ones_v, acc_sh,
                   s0, s1):
        cid = lax.axis_index("c")
        sid = lax.axis_index("s")
        wid = cid * NS + sid
        pltpu.sync_copy(edge3d.at[1, pl.ds(wid * CH, CH)], dst_v)
        pltpu.sync_copy(ones_hbm, ones_v)
        pltpu.sync_copy(zeros_hbm.at[pl.ds(sid * RPT, RPT)],
                        acc_sh.at[pl.ds(sid * RPT, RPT)])
        plsc.subcore_barrier()

        def s_start(j, sem):
            pltpu.async_copy(ones_v, acc_sh.at[dst_v.at[j]], sem, add=True)

        def s_wait(j, sem):
            pltpu.make_async_copy(ones_v, acc_sh.at[dst_v.at[j]], sem).wait()

        s_start(0, s0)
        s_start(1, s1)

        def body(i, carry):
            j0 = 2 * i
            s_wait(j0, s0)
            s_start(j0 + 2, s0)
            s_wait(j0 + 1, s1)
            s_start(j0 + 3, s1)
            return carry

        lax.fori_loop(0, CH // 2 - 1, body, 0)
        s_wait(CH - 2, s0)
        s_wait(CH - 1, s1)
        plsc.subcore_barrier()
        pltpu.sync_copy(acc_sh.at[pl.ds(sid * RPT, RPT)],
                        out_hbm.at[pl.ds(sid * RPT, RPT),
                                   pl.ds(cid * DEG_W, DEG_W)])

    return deg_kernel


def _make_seg_kernel(D, DEPTH):
    @functools.partial(
        pl.kernel,
        out_type=jax.ShapeDtypeStruct((ACC_N, 128), jnp.float32),
        scratch_types=[
            pltpu.VMEM((CH, C), jnp.int32),
            pltpu.VMEM((CH, C), jnp.int32),
        ] + [pltpu.VMEM((C, D), jnp.float32) for _ in range(DEPTH)] + [
            pltpu.VMEM_SHARED((ACC_N, D), jnp.float32),
            pltpu.VMEM_SHARED((ACC_N, D), jnp.float32),
        ] + [pltpu.SemaphoreType.DMA for _ in range(2 * DEPTH)],
        **_SC_PARAMS,
    )
    def seg_kernel(y_hbm, edge3d, zeros_hbm, out_hbm, src_v, dst_v, *rest):
        bufs = rest[:DEPTH]
        acc_sh, tab_sh = rest[DEPTH], rest[DEPTH + 1]
        gsems = rest[DEPTH + 2:2 * DEPTH + 2]
        ssems = rest[2 * DEPTH + 2:]
        cid = lax.axis_index("c")
        sid = lax.axis_index("s")
        wid = cid * NS + sid
        pltpu.sync_copy(edge3d.at[0, pl.ds(wid * CH, CH)], src_v)
        pltpu.sync_copy(edge3d.at[1, pl.ds(wid * CH, CH)], dst_v)
        pltpu.sync_copy(zeros_hbm.at[pl.ds(sid * RPT, RPT)],
                        acc_sh.at[pl.ds(sid * RPT, RPT)])
        pltpu.sync_copy(y_hbm.at[pl.ds(sid * RPT, RPT), pl.ds(0, D)],
                        tab_sh.at[pl.ds(sid * RPT, RPT)])
        plsc.subcore_barrier()

        def g_start(j, buf, sem):
            pltpu.async_copy(tab_sh.at[src_v.at[j]], buf, sem)

        def g_wait(j, buf, sem):
            pltpu.make_async_copy(tab_sh.at[src_v.at[j]], buf, sem).wait()

        def s_start(j, buf, sem):
            pltpu.async_copy(buf, acc_sh.at[dst_v.at[j]], sem, add=True)

        def s_wait(j, buf, sem):
            pltpu.make_async_copy(buf, acc_sh.at[dst_v.at[j]], sem).wait()

        for k in range(DEPTH):
            g_start(k, bufs[k], gsems[k])

        def body(i, carry):
            j0 = DEPTH * i
            for k in range(DEPTH):
                g_wait(j0 + k, bufs[k], gsems[k])
                s_start(j0 + k, bufs[k], ssems[k])
            for k in range(DEPTH):
                s_wait(j0 + k, bufs[k], ssems[k])
                g_start(j0 + DEPTH + k, bufs[k], gsems[k])
            return carry

        lax.fori_loop(0, CH // DEPTH - 1, body, 0)
        j0 = CH - DEPTH
        for k in range(DEPTH):
            g_wait(j0 + k, bufs[k], gsems[k])
            s_start(j0 + k, bufs[k], ssems[k])
        for k in range(DEPTH):
            s_wait(j0 + k, bufs[k], ssems[k])
        plsc.subcore_barrier()
        pltpu.sync_copy(acc_sh.at[pl.ds(sid * RPT, RPT)],
                        out_hbm.at[pl.ds(sid * RPT, RPT), pl.ds(cid * D, D)])

    return seg_kernel


_deg_kernel = _make_deg_kernel()
_seg_kernel_h = _make_seg_kernel(D_H, 2)
_seg_kernel_o = _make_seg_kernel(D_OUT, 4)


BN = 1000
G = N // BN


def _mm_body(x_ref, w_ref, o_ref):
    o_ref[...] = jnp.dot(x_ref[...], w_ref[...],
                         preferred_element_type=jnp.float32)


def _scale_body(xw_ref, degc_ref, y_ref, dis_ref):
    d = degc_ref[...]
    deg = d[:, 0:1] + d[:, DEG_W:DEG_W + 1] + 1.0
    dis = lax.rsqrt(deg)
    dis_ref[...] = dis
    y_ref[...] = jnp.concatenate(
        [dis * xw_ref[...], jnp.zeros((BN, 128 - D_H), jnp.float32)], axis=1)


def _tc_b_body(a_ref, xw_ref, dis_ref, w_ref, b_ref, y2_ref, y2n_ref):
    dis = dis_ref[...]
    ac = a_ref[...]
    a = ac[:, 0:D_H] + ac[:, D_H:2 * D_H]
    h = dis * (a + dis * xw_ref[...]) + b_ref[...]
    h = jnp.maximum(h, 0.0)
    y2 = dis * jnp.dot(h, w_ref[...], preferred_element_type=jnp.float32)
    y2_ref[...] = jnp.concatenate(
        [y2, jnp.zeros((BN, 128 - D_OUT), jnp.float32)], axis=1)
    y2n_ref[...] = y2


def _tc_c_body(a_ref, y2_ref, dis_ref, b_ref, out_ref):
    ac = a_ref[...]
    a = ac[:, 0:D_OUT] + ac[:, D_OUT:2 * D_OUT]
    out_ref[...] = dis_ref[...] * (a + y2_ref[...]) + b_ref[...]


def _row_spec(d):
    return pl.BlockSpec((BN, d), lambda i: (i, 0))


def _full_spec(shape):
    return pl.BlockSpec(shape, lambda i: (0,) * len(shape))


def _tc_mm(x, W1):
    return pl.pallas_call(
        _mm_body,
        grid=(G,),
        in_specs=[_row_spec(D_IN), _full_spec((D_IN, D_H))],
        out_specs=_row_spec(D_H),
        out_shape=jax.ShapeDtypeStruct((N, D_H), jnp.float32),
    )(x, W1)


def _tc_scale(xw, degc):
    return pl.pallas_call(
        _scale_body,
        grid=(G,),
        in_specs=[_row_spec(D_H), _row_spec(128)],
        out_specs=[_row_spec(128), _row_spec(1)],
        out_shape=[jax.ShapeDtypeStruct((ACC_N, 128), jnp.float32),
                   jax.ShapeDtypeStruct((N, 1), jnp.float32)],
    )(xw, degc)


def _tc_b(acc1, xw, dis, W2, b1):
    return pl.pallas_call(
        _tc_b_body,
        grid=(G,),
        in_specs=[_row_spec(128), _row_spec(D_H), _row_spec(1),
                  _full_spec((D_H, D_OUT)), _full_spec((1, D_H))],
        out_specs=[_row_spec(128), _row_spec(D_OUT)],
        out_shape=[jax.ShapeDtypeStruct((ACC_N, 128), jnp.float32),
                   jax.ShapeDtypeStruct((N, D_OUT), jnp.float32)],
    )(acc1, xw, dis, W2, b1)


def _tc_c(acc2, y2, dis, b2):
    return pl.pallas_call(
        _tc_c_body,
        grid=(G,),
        in_specs=[_row_spec(128), _row_spec(D_OUT), _row_spec(1),
                  _full_spec((1, D_OUT))],
        out_specs=_row_spec(D_OUT),
        out_shape=jax.ShapeDtypeStruct((N, D_OUT), jnp.float32),
    )(acc2, y2, dis, b2)


def kernel(x, edge_index, W1, b1, W2, b2):
    edge3d = edge_index.reshape(2, ROWS, C)

    ones_rows = jnp.ones((C, DEG_W), jnp.float32)
    zeros16 = jnp.zeros((ACC_N, DEG_W), jnp.float32)
    zeros64 = jnp.zeros((ACC_N, D_H), jnp.float32)
    zeros32 = jnp.zeros((ACC_N, D_OUT), jnp.float32)

    degc = _deg_kernel(edge3d, ones_rows, zeros16)
    xw = _tc_mm(x, W1)
    y1, dis = _tc_scale(xw, degc)

    acc1 = _seg_kernel_h(y1, edge3d, zeros64)
    y2w, y2n = _tc_b(acc1, xw, dis, W2, b1.reshape(1, D_H))

    acc2 = _seg_kernel_o(y2w, edge3d, zeros32)
    return _tc_c(acc2, y2n, dis, b2.reshape(1, D_OUT))
